# baseline (speedup 1.0000x reference)
import functools

import jax
import jax.numpy as jnp
from jax import lax
from jax.experimental import pallas as pl
from jax.experimental.pallas import tpu as pltpu
from jax.experimental.pallas import tpu_sc as plsc

_L = 16
_B = 97
_N_FULL = 6


def _body(xf_hbm, xw_hbm, xs_hbm, e1_hbm, e2_hbm, w0_hbm, b0_hbm, w1_hbm,
          b1_hbm, w2_hbm, b2_hbm, out_hbm,
          e1_v, e2_v, w0_v, b0_v, w1_v, b1_v, wb_v,
          xf_v, xw_v, xs_v, z_v, out_v, sem):
    wid = lax.axis_index("s")

    @pl.when(wid <= _N_FULL)
    def _():
        cps = [pltpu.async_copy(s, d, sem) for s, d in (
            (e1_hbm, e1_v), (e2_hbm, e2_v), (w0_hbm, w0_v), (b0_hbm, b0_v),
            (w1_hbm, w1_v), (b1_hbm, b1_v),
            (w2_hbm, wb_v.at[pl.ds(8, 8)]), (b2_hbm, wb_v.at[pl.ds(16, 8)]),
        )]

        for s, d in ((xf_hbm, xf_v), (xw_hbm, xw_v), (xs_hbm, xs_v)):
            cps.append(pltpu.async_copy(
                s.at[pl.ds(pl.multiple_of(wid * _L, _L), _L)], d, sem))

        for cp in cps:
            cp.wait()

        lanes = lax.iota(jnp.int32, _L)
        col4 = lax.rem(lanes, 4)

        def splat(v):
            return jnp.full((_L,), v, jnp.int32)

        def off(x, c):
            return x if c == 0 else x + c

        for half in range(2):
            flat = off(lanes, half * _L)
            row1 = lax.div(flat, 4)
            row2 = lax.min(row1, splat(4))
            acc1 = plsc.load_gather(b0_v, [col4])
            acc2 = plsc.load_gather(b1_v, [col4])
            for k in range(3):
                a1 = plsc.load_gather(e1_v, [off(row1 * 3, k)])
                w0 = plsc.load_gather(w0_v, [off(col4, k * 4)])
                acc1 = acc1 + a1 * w0
                a2 = plsc.load_gather(e2_v, [off(row2 * 3, k)])
                w1 = plsc.load_gather(w1_v, [off(col4, k * 4)])
                acc2 = acc2 + a2 * w1
            z_v[pl.ds(half * _L, _L)] = acc1
            z_v[pl.ds(32 + half * _L, _L)] = acc2

        ft = xf_v[...]
        wk = xw_v[...]
        st = xs_v[...]
        rowbase = lanes * _L

        for j in range(8):
            colv = ft * plsc.load_gather(wb_v, [splat(8 + j)]) \
                + plsc.load_gather(wb_v, [splat(16 + j)])
            plsc.store_scatter(out_v, [off(rowbase, j)], colv)
        for j in range(4):
            plsc.store_scatter(out_v, [off(rowbase, 8 + j)],
                               plsc.load_gather(z_v, [off(st * 4, 32 + j)]))
            plsc.store_scatter(out_v, [off(rowbase, 12 + j)],
                               plsc.load_gather(z_v, [off(wk * 4, j)]))

        @pl.when(wid < _N_FULL)
        def _():
            pltpu.sync_copy(
                out_v,
                out_hbm.at[pl.ds(pl.multiple_of(wid * _L * _L, _L * _L),
                                 _L * _L)])

        @pl.when(wid == _N_FULL)
        def _():
            pltpu.sync_copy(out_v.at[pl.ds(0, _L)],
                            out_hbm.at[pl.ds(_N_FULL * _L * _L, _L)])


@functools.partial(
    pl.kernel,
    mesh=plsc.VectorSubcoreMesh(core_axis_name="c", subcore_axis_name="s",
                                num_cores=1),
    out_type=jax.ShapeDtypeStruct((_B * _L,), jnp.float32),
    compiler_params=pltpu.CompilerParams(needs_layout_passes=False,
                                         skip_device_barrier=True),
    scratch_types=[
        pltpu.VMEM((24,), jnp.float32),
        pltpu.VMEM((15,), jnp.float32),
        pltpu.VMEM((12,), jnp.float32),
        pltpu.VMEM((4,), jnp.float32),
        pltpu.VMEM((12,), jnp.float32),
        pltpu.VMEM((4,), jnp.float32),
        pltpu.VMEM((24,), jnp.float32),
        pltpu.VMEM((_L,), jnp.float32),
        pltpu.VMEM((_L,), jnp.int32),
        pltpu.VMEM((_L,), jnp.int32),
        pltpu.VMEM((64,), jnp.float32),
        pltpu.VMEM((_L * _L,), jnp.float32),
        pltpu.SemaphoreType.DMA,
    ],
)
def _sc_kernel(*refs):
    _body(*refs)


def kernel(X_feature, X_week, X_stamp, emb1, emb2, W0, b0, W1, b1, W2, b2):
    f32 = jnp.float32
    pad = _N_FULL * _L + _L - _B
    out = _sc_kernel(
        jnp.pad(X_feature.astype(f32), (0, pad)),
        jnp.pad(X_week.astype(jnp.int32), (0, pad)),
        jnp.pad(X_stamp.astype(jnp.int32), (0, pad)),
        emb1.reshape(-1).astype(f32),
        emb2.reshape(-1).astype(f32),
        W0.reshape(-1).astype(f32),
        b0.astype(f32),
        W1.reshape(-1).astype(f32),
        b1.astype(f32),
        W2.reshape(-1).astype(f32),
        b2.astype(f32),
    )
    return out.reshape(_B, _L)
